# Initial kernel scaffold; baseline (speedup 1.0000x reference)
import jax
import jax.numpy as jnp
from jax.experimental import pallas as pl


def kernel(x, edge_index, edge_weights, batch, W, b):
    raise NotImplementedError("write your pallas kernel here")


# speedup vs baseline: 34.9628x; 34.9628x over previous
import functools

import jax
import jax.numpy as jnp
from jax import lax
from jax.experimental import pallas as pl
from jax.experimental.pallas import tpu as pltpu
from jax.experimental.pallas import tpu_sc as plsc

NP = 10240
EP = 163840
NP2 = 12288
GP = 256
PAD_NODE = 10200
PAD_GRAPH = 200
NG = 128

_MESH = plsc.VectorSubcoreMesh(core_axis_name="c", subcore_axis_name="s")


def _zero_fill(ref, nwords):
    z = jnp.zeros((16,), ref.dtype)

    def body(i, carry):
        ref[pl.ds(i * 16, 16)] = z
        return carry

    lax.fori_loop(0, nwords // 16, body, 0)


@functools.partial(
    pl.kernel,
    out_type=(
        jax.ShapeDtypeStruct((2, NP), jnp.float32),
        jax.ShapeDtypeStruct((2, GP), jnp.float32),
    ),
    mesh=_MESH,
    compiler_params=pltpu.CompilerParams(needs_layout_passes=False),
    scratch_types=[
        pltpu.VMEM((40, 128), jnp.int32),
        pltpu.VMEM((40, 128), jnp.float32),
        pltpu.VMEM((3, 128), jnp.int32),
        pltpu.VMEM((3, 128), jnp.float32),
        pltpu.VMEM((640,), jnp.float32),
        pltpu.VMEM_SHARED((NP,), jnp.float32),
        pltpu.VMEM_SHARED((GP,), jnp.float32),
    ],
)
def _deg_counts_sc(dst3, ew3, bat3, degp, cntp,
                   dstv, ewv, batv, onesv, zb, degsh, cntsh):
    c = lax.axis_index("c")
    s = lax.axis_index("s")
    wid = s * 2 + c

    _zero_fill(zb, 640)
    one = jnp.ones((16,), jnp.float32)
    for j in range(3):
        for k in range(8):
            onesv[j, pl.ds(k * 16, 16)] = one

    pltpu.sync_copy(zb, degsh.at[pl.ds(s * 640, 640)])

    @pl.when(s == 0)
    def _():
        pltpu.sync_copy(zb.at[pl.ds(0, GP)], cntsh)

    plsc.subcore_barrier()

    pltpu.sync_copy(dst3.at[wid], dstv)
    pltpu.sync_copy(ew3.at[wid], ewv)
    pltpu.sync_copy(bat3.at[wid], batv)

    def edge_batch(j, carry):
        pltpu.sync_copy(ewv.at[j], degsh.at[dstv.at[j]], add=True)
        return carry

    lax.fori_loop(0, 40, edge_batch, 0)

    for j in range(3):
        pltpu.sync_copy(onesv.at[j], cntsh.at[batv.at[j]], add=True)

    plsc.subcore_barrier()

    pltpu.sync_copy(degsh.at[pl.ds(s * 640, 640)],
                    degp.at[c, pl.ds(s * 640, 640)])

    @pl.when(s == 1)
    def _():
        pltpu.sync_copy(cntsh, cntp.at[c])


_GH = NG // 2
_DEAD = _GH * NP
_SWORDS = _GH * NP + 128
_TILE_S = 40960


@functools.partial(
    pl.kernel,
    out_type=jax.ShapeDtypeStruct((2, _SWORDS), jnp.float32),
    mesh=_MESH,
    compiler_params=pltpu.CompilerParams(needs_layout_passes=False),
    scratch_types=[
        pltpu.VMEM((80, 128), jnp.int32),
        pltpu.VMEM((80, 128), jnp.int32),
        pltpu.VMEM((80, 128), jnp.float32),
        pltpu.VMEM((6, 128), jnp.int32),
        pltpu.VMEM((NP,), jnp.int32),
        pltpu.VMEM((NP,), jnp.float32),
        pltpu.VMEM((NP,), jnp.float32),
        pltpu.VMEM((128,), jnp.int32),
        pltpu.VMEM((128,), jnp.float32),
        pltpu.VMEM((8192,), jnp.float32),
        pltpu.VMEM_SHARED((_SWORDS,), jnp.float32),
    ],
)
def _smat_sc(src3, dst3, ew3, nid3, batf, disf, invf, sp,
             srcv, dstv, ewv, nidv, batv, disv, invv, idxb, valb, zb, ssh):
    c = lax.axis_index("c")
    s = lax.axis_index("s")
    glo = c * _GH

    _zero_fill(zb, 8192)

    def zloop(i, carry):
        pltpu.sync_copy(zb, ssh.at[pl.ds(s * _TILE_S + i * 8192, 8192)])
        return carry

    lax.fori_loop(0, _TILE_S // 8192, zloop, 0)

    @pl.when(s == 0)
    def _():
        pltpu.sync_copy(zb.at[pl.ds(0, 128)], ssh.at[pl.ds(16 * _TILE_S, 128)])

    pltpu.sync_copy(src3.at[s], srcv)
    pltpu.sync_copy(dst3.at[s], dstv)
    pltpu.sync_copy(ew3.at[s], ewv)
    pltpu.sync_copy(nid3.at[s], nidv)
    pltpu.sync_copy(batf, batv)
    pltpu.sync_copy(disf, disv)
    pltpu.sync_copy(invf, invv)

    plsc.subcore_barrier()

    lanes = lax.iota(jnp.int32, 16)

    def edge_batch(j, carry):
        def sub(k, carry2):
            s16 = srcv[j, pl.ds(k * 16, 16)]
            d16 = dstv[j, pl.ds(k * 16, 16)]
            w16 = ewv[j, pl.ds(k * 16, 16)]
            ds_ = plsc.load_gather(disv, [d16])
            ss_ = plsc.load_gather(disv, [s16])
            g16 = plsc.load_gather(batv, [d16]) - glo
            own = (g16 >= 0) & (g16 < _GH)
            dead = _DEAD + ((lanes + k * 16) & 127)
            idxb[pl.ds(k * 16, 16)] = jnp.where(own, g16 * NP + s16, dead)
            valb[pl.ds(k * 16, 16)] = jnp.where(own, ds_ * ss_ * w16, 0.0)
            return carry2

        lax.fori_loop(0, 8, sub, 0)
        pltpu.sync_copy(valb, ssh.at[idxb], add=True)
        return carry

    lax.fori_loop(0, 80, edge_batch, 0)

    def self_batch(j, carry):
        def sub(k, carry2):
            n16 = nidv[j, pl.ds(k * 16, 16)]
            g16 = plsc.load_gather(batv, [n16]) - glo
            v16 = plsc.load_gather(invv, [n16])
            own = (g16 >= 0) & (g16 < _GH)
            dead = _DEAD + ((lanes + k * 16) & 127)
            idxb[pl.ds(k * 16, 16)] = jnp.where(own, g16 * NP + n16, dead)
            valb[pl.ds(k * 16, 16)] = jnp.where(own, v16, 0.0)
            return carry2

        lax.fori_loop(0, 8, sub, 0)
        pltpu.sync_copy(valb, ssh.at[idxb], add=True)
        return carry

    lax.fori_loop(0, 6, self_batch, 0)

    plsc.subcore_barrier()

    pltpu.sync_copy(ssh.at[pl.ds(s * _TILE_S, _TILE_S)],
                    sp.at[c, pl.ds(s * _TILE_S, _TILE_S)])

    @pl.when(s == 1)
    def _():
        pltpu.sync_copy(ssh.at[pl.ds(16 * _TILE_S, 128)],
                        sp.at[c, pl.ds(16 * _TILE_S, 128)])


def _prep_tc_body(degp_ref, cntp_ref, di_ref, cnt_ref):
    deg = degp_ref[0:1, :] + degp_ref[1:2, :] + 1.0
    di_ref[0:1, :] = lax.rsqrt(deg)
    di_ref[1:2, :] = 1.0 / deg
    cnt_ref[...] = cntp_ref[0:1, :] + cntp_ref[1:2, :]


_BN = 1024


def _gcn_tc_body(xb_ref, w_ref, sb_ref, cnt_ref, b_ref, out_ref, acc_ref):
    k = pl.program_id(0)

    @pl.when(k == 0)
    def _():
        acc_ref[...] = jnp.zeros_like(acc_ref)

    h = jnp.dot(xb_ref[...], w_ref[...], preferred_element_type=jnp.float32)
    acc_ref[...] += jnp.dot(sb_ref[...], h, preferred_element_type=jnp.float32)

    @pl.when(k == pl.num_programs(0) - 1)
    def _():
        p = acc_ref[...]
        cnt = cnt_ref[...]
        pooled = (p + cnt * b_ref[...]) / jnp.maximum(cnt, 1.0)
        m = jnp.max(pooled, axis=1, keepdims=True)
        shifted = pooled - m
        out_ref[...] = shifted - jnp.log(
            jnp.sum(jnp.exp(shifted), axis=1, keepdims=True))


def kernel(x, edge_index, edge_weights, batch, W, b):
    n, d = x.shape
    cdim = W.shape[1]
    e = edge_index.shape[1]

    src = edge_index[0]
    dst = edge_index[1]

    pad_e = EP - e
    pad_i = jnp.full((pad_e,), PAD_NODE, jnp.int32)
    srcp = jnp.concatenate([src, pad_i])
    dstp = jnp.concatenate([dst, pad_i])
    ewp = jnp.concatenate([edge_weights, jnp.zeros((pad_e,), jnp.float32)])

    batf = jnp.concatenate([batch, jnp.zeros((NP - n,), jnp.int32)])
    bat3 = jnp.concatenate(
        [batch, jnp.full((NP2 - n,), PAD_GRAPH, jnp.int32)]).reshape(32, 3, 128)
    nid3 = jnp.concatenate(
        [jnp.arange(n, dtype=jnp.int32),
         jnp.full((NP2 - n,), PAD_NODE, jnp.int32)]).reshape(16, 6, 128)

    degp, cntp = _deg_counts_sc(dstp.reshape(32, 40, 128),
                                ewp.reshape(32, 40, 128), bat3)

    di, cnt = pl.pallas_call(
        _prep_tc_body,
        out_shape=(
            jax.ShapeDtypeStruct((2, NP), jnp.float32),
            jax.ShapeDtypeStruct((1, GP), jnp.float32),
        ),
    )(degp, cntp)

    sp = _smat_sc(srcp.reshape(16, 80, 128), dstp.reshape(16, 80, 128),
                  ewp.reshape(16, 80, 128), nid3, batf, di[0], di[1])
    s2 = sp[:, :_GH * NP].reshape(NG, NP)

    xp = jnp.pad(x, ((0, NP - n), (0, 0)))
    cnt_col = cnt[0, :NG].reshape(NG, 1)
    b_row = b.reshape(1, cdim)

    out = pl.pallas_call(
        _gcn_tc_body,
        grid=(NP // _BN,),
        in_specs=[
            pl.BlockSpec((_BN, d), lambda k: (k, 0)),
            pl.BlockSpec((d, cdim), lambda k: (0, 0)),
            pl.BlockSpec((NG, _BN), lambda k: (0, k)),
            pl.BlockSpec((NG, 1), lambda k: (0, 0)),
            pl.BlockSpec((1, cdim), lambda k: (0, 0)),
        ],
        out_specs=pl.BlockSpec((NG, cdim), lambda k: (0, 0)),
        out_shape=jax.ShapeDtypeStruct((NG, cdim), jnp.float32),
        scratch_shapes=[pltpu.VMEM((NG, cdim), jnp.float32)],
    )(xp, W, s2, cnt_col, b_row)

    return out
